# Initial kernel scaffold; baseline (speedup 1.0000x reference)
import jax
import jax.numpy as jnp
from jax.experimental import pallas as pl


def kernel(x, edge_index, edge_attr, We1, W1, b1, We2, W2, b2):
    raise NotImplementedError("write your pallas kernel here")


# speedup vs baseline: 5.2179x; 5.2179x over previous
import functools

import jax
import jax.numpy as jnp
from jax import lax
from jax.experimental import pallas as pl
from jax.experimental.pallas import tpu as pltpu
from jax.experimental.pallas import tpu_sc as plsc

_N = 10000
_NP = 10240
_E = 320000
_NC = 2
_NS = 16
_NW = _NC * _NS
_K = 80
_EPW = _E // _NW
_CH = _EPW // _K
_RPS = _NP // _NS
_D1P = 80
_DE = 16
_D2P = 16

_f32 = jnp.float32


def _sc_mesh():
    return plsc.VectorSubcoreMesh(core_axis_name="c", subcore_axis_name="s")


def _zero_rows(zb, ncols):
    @pl.loop(0, zb.shape[0])
    def _(r):
        for c in range(ncols // 16):
            zb[r, pl.ds(c * 16, 16)] = jnp.zeros((16,), _f32)


def _scatter_layer1(y1, ei4):
    @functools.partial(
        pl.kernel,
        mesh=_sc_mesh(),
        compiler_params=pltpu.CompilerParams(use_tc_tiling_on_sc=False),
        out_type=[jax.ShapeDtypeStruct((_NC, _NP, _D1P), _f32)],
        scratch_types=[
            pltpu.VMEM((_CH, _K), jnp.int32),
            pltpu.VMEM((_CH, _K), jnp.int32),
            pltpu.VMEM((_K, _D1P), _f32),
            pltpu.VMEM((_RPS, _D1P), _f32),
            pltpu.VMEM_SHARED((_NP, _D1P), _f32),
            pltpu.SemaphoreType.DMA,
        ],
    )
    def body(y1_hbm, ei_hbm, s1_hbm, srcv, dstv, rows, zb1, acc1, sem):
        cid = lax.axis_index("c")
        sid = lax.axis_index("s")
        wid = sid * _NC + cid
        rbase = sid * _RPS

        _zero_rows(zb1, _D1P)
        pltpu.sync_copy(zb1, acc1.at[pl.ds(rbase, _RPS)])
        pltpu.sync_copy(ei_hbm.at[0, wid], srcv)
        pltpu.sync_copy(ei_hbm.at[1, wid], dstv)
        plsc.subcore_barrier()

        @pl.loop(0, _CH)
        def _(j):
            pltpu.async_copy(y1_hbm.at[srcv.at[j]], rows, sem).wait()
            pltpu.sync_copy(rows, acc1.at[dstv.at[j]], add=True)

        plsc.subcore_barrier()
        pltpu.sync_copy(acc1.at[pl.ds(rbase, _RPS)],
                        s1_hbm.at[cid, pl.ds(rbase, _RPS)])

    return body(y1, ei4)[0]


def _scatter_edges(eap, ei4):
    @functools.partial(
        pl.kernel,
        mesh=_sc_mesh(),
        compiler_params=pltpu.CompilerParams(use_tc_tiling_on_sc=False),
        out_type=[jax.ShapeDtypeStruct((_NC, _NP, _DE), _f32)],
        scratch_types=[
            pltpu.VMEM((_CH, _K), jnp.int32),
            pltpu.VMEM((_K, _DE), _f32),
            pltpu.VMEM((_RPS, _DE), _f32),
            pltpu.VMEM_SHARED((_NP, _DE), _f32),
            pltpu.SemaphoreType.DMA,
        ],
    )
    def body(eap_hbm, ei_hbm, ec_hbm, dstv, earows, zb2, acc2, sem):
        cid = lax.axis_index("c")
        sid = lax.axis_index("s")
        wid = sid * _NC + cid
        rbase = sid * _RPS

        _zero_rows(zb2, _DE)
        pltpu.sync_copy(zb2, acc2.at[pl.ds(rbase, _RPS)])
        pltpu.sync_copy(ei_hbm.at[1, wid], dstv)
        plsc.subcore_barrier()

        @pl.loop(0, _CH)
        def _(j):
            pltpu.sync_copy(
                eap_hbm.at[pl.ds(wid * _EPW + j * _K, _K)], earows)
            pltpu.sync_copy(earows, acc2.at[dstv.at[j]], add=True)

        plsc.subcore_barrier()
        pltpu.sync_copy(acc2.at[pl.ds(rbase, _RPS)],
                        ec_hbm.at[cid, pl.ds(rbase, _RPS)])

    return body(eap, ei4)[0]


def _scatter_layer2(y2, ei4):
    @functools.partial(
        pl.kernel,
        mesh=_sc_mesh(),
        compiler_params=pltpu.CompilerParams(use_tc_tiling_on_sc=False),
        out_type=[jax.ShapeDtypeStruct((_NC, _NP, _D2P), _f32)],
        scratch_types=[
            pltpu.VMEM((_CH, _K), jnp.int32),
            pltpu.VMEM((_CH, _K), jnp.int32),
            pltpu.VMEM((_K, _D2P), _f32),
            pltpu.VMEM((_RPS, _D2P), _f32),
            pltpu.VMEM_SHARED((_NP, _D2P), _f32),
            pltpu.SemaphoreType.DMA,
        ],
    )
    def body(y2_hbm, ei_hbm, s2_hbm, srcv, dstv, rows, zb, acc, sem):
        cid = lax.axis_index("c")
        sid = lax.axis_index("s")
        wid = sid * _NC + cid
        rbase = sid * _RPS

        _zero_rows(zb, _D2P)
        pltpu.sync_copy(zb, acc.at[pl.ds(rbase, _RPS)])
        pltpu.sync_copy(ei_hbm.at[0, wid], srcv)
        pltpu.sync_copy(ei_hbm.at[1, wid], dstv)
        plsc.subcore_barrier()

        @pl.loop(0, _CH)
        def _(j):
            pltpu.async_copy(y2_hbm.at[srcv.at[j]], rows, sem).wait()
            pltpu.sync_copy(rows, acc.at[dstv.at[j]], add=True)

        plsc.subcore_barrier()
        pltpu.sync_copy(acc.at[pl.ds(rbase, _RPS)],
                        s2_hbm.at[cid, pl.ds(rbase, _RPS)])

    return body(y2, ei4)[0]


def _tc_edge_pad(edge_attr):
    blk = 8000
    def body(ea_ref, o_ref):
        ea = ea_ref[...]
        o_ref[...] = jnp.concatenate(
            [ea,
             jnp.ones((blk, 1), _f32),
             jnp.zeros((blk, _DE - 3), _f32)], axis=1)
    return pl.pallas_call(
        body,
        grid=(_E // blk,),
        in_specs=[pl.BlockSpec((blk, 2), lambda i: (i, 0))],
        out_specs=pl.BlockSpec((blk, _DE), lambda i: (i, 0)),
        out_shape=jax.ShapeDtypeStruct((_E, _DE), _f32))(edge_attr)


def _tc_in_matmul(x, w1):
    def body(x_ref, w_ref, o_ref):
        w = jnp.concatenate(
            [w_ref[...], jnp.zeros((w_ref.shape[0], _D1P - w_ref.shape[1]),
                                   _f32)], axis=1)
        y = jnp.dot(x_ref[...], w, preferred_element_type=_f32)
        o_ref[...] = jnp.concatenate(
            [y, jnp.zeros((_NP - _N, _D1P), _f32)], axis=0)
    return pl.pallas_call(
        body, out_shape=jax.ShapeDtypeStruct((_NP, _D1P), _f32))(x, w1)


def _tc_epilogue1(s1p, ecp, we1, w1, b1, w2):
    def body(s_ref, ec_ref, we_ref, w1_ref, b_ref, w2_ref, o_ref):
        s = s_ref[0] + s_ref[1]
        ec = ec_ref[0] + ec_ref[1]
        w1p = jnp.concatenate(
            [w1_ref[...], jnp.zeros((128, _D1P - 69), _f32)], axis=1)
        we = jnp.concatenate(
            [we_ref[...], jnp.zeros((_DE - 2, 128), _f32)], axis=0)
        t = jnp.dot(we, w1p, preferred_element_type=_f32)
        lane = lax.broadcasted_iota(jnp.int32, (_NP, _DE), 1)
        ecz = jnp.where(lane == 2, 0.0, ec)
        cterm = jnp.dot(ecz, t, preferred_element_type=_f32)
        deg = jnp.maximum(ec[:, 2:3], 1.0)
        bp = jnp.concatenate(
            [b_ref[...], jnp.zeros((1, _D1P - 69), _f32)], axis=1)
        h = jnp.maximum((s + cterm) / deg + bp, 0.0)
        w2p = jnp.concatenate(
            [jnp.concatenate([w2_ref[...], jnp.zeros((69, _D2P - 10), _f32)],
                             axis=1),
             jnp.zeros((_D1P - 69, _D2P), _f32)], axis=0)
        o_ref[...] = jnp.dot(h, w2p, preferred_element_type=_f32)

    return pl.pallas_call(
        body, out_shape=jax.ShapeDtypeStruct((_NP, _D2P), _f32))(
            s1p, ecp, we1, w1, b1.reshape(1, 69), w2)


def _tc_epilogue2(s2p, ecp, we2, w2, b2):
    def body(s_ref, ec_ref, we_ref, w2_ref, b_ref, o_ref):
        s = s_ref[0] + s_ref[1]
        ec = ec_ref[0] + ec_ref[1]
        we = jnp.concatenate(
            [we_ref[...], jnp.zeros((_DE - 2, 69), _f32)], axis=0)
        t = jnp.dot(we, w2_ref[...], preferred_element_type=_f32)
        lane = lax.broadcasted_iota(jnp.int32, (_NP, _DE), 1)
        ecz = jnp.where(lane == 2, 0.0, ec)
        cterm = jnp.dot(ecz, t, preferred_element_type=_f32)
        deg = jnp.maximum(ec[:, 2:3], 1.0)
        h = jnp.maximum((s[:, :10] + cterm) / deg + b_ref[...], 0.0)
        o_ref[...] = h[:_N]

    return pl.pallas_call(
        body, out_shape=jax.ShapeDtypeStruct((_N, 10), _f32))(
            s2p, ecp, we2, w2, b2.reshape(1, 10))


def kernel(x, edge_index, edge_attr, We1, W1, b1, We2, W2, b2):
    ei4 = edge_index.reshape(2, _NW, _CH, _K)
    eap = _tc_edge_pad(edge_attr)
    y1 = _tc_in_matmul(x, W1)
    s1p = _scatter_layer1(y1, ei4)
    ecp = _scatter_edges(eap, ei4)
    y2 = _tc_epilogue1(s1p, ecp, We1, W1, b1, W2)
    s2p = _scatter_layer2(y2, ei4)
    return _tc_epilogue2(s2p, ecp, We2, W2, b2)


# speedup vs baseline: 7.5921x; 1.4550x over previous
import functools

import jax
import jax.numpy as jnp
from jax import lax
from jax.experimental import pallas as pl
from jax.experimental.pallas import tpu as pltpu
from jax.experimental.pallas import tpu_sc as plsc

_N = 10000
_NP = 10240
_E = 320000
_NC = 2
_NS = 16
_NW = _NC * _NS
_K = 80
_EPW = _E // _NW
_CH = _EPW // _K
_RPS = _NP // _NS
_D1P = 80
_DE = 16
_D2P = 16

_f32 = jnp.float32


def _sc_mesh():
    return plsc.VectorSubcoreMesh(core_axis_name="c", subcore_axis_name="s")


def _zero_rows(zb, ncols):
    @pl.loop(0, zb.shape[0])
    def _(r):
        for c in range(ncols // 16):
            zb[r, pl.ds(c * 16, 16)] = jnp.zeros((16,), _f32)


def _sc_scatter(table, ei4, d, gather):
    NB = 4
    scratch = [pltpu.VMEM((_CH, _K), jnp.int32)]
    if gather:
        scratch.append(pltpu.VMEM((_CH, _K), jnp.int32))
    scratch += [pltpu.VMEM((_K, d), _f32) for _ in range(NB)]
    scratch += [pltpu.VMEM_SHARED((_NP, d), _f32)]
    scratch += [pltpu.SemaphoreType.DMA] * (2 * NB)

    @functools.partial(
        pl.kernel,
        mesh=_sc_mesh(),
        compiler_params=pltpu.CompilerParams(use_tc_tiling_on_sc=False),
        out_type=[jax.ShapeDtypeStruct((_NC, _NP, d), _f32)],
        scratch_types=scratch,
    )
    def body(tab_hbm, ei_hbm, out_hbm, *rest):
        dstv = rest[0]
        rest = rest[1:]
        if gather:
            srcv = rest[0]
            rest = rest[1:]
        bufs = rest[:NB]
        acc = rest[NB]
        gsem = rest[NB + 1:NB + 1 + NB]
        ssem = rest[NB + 1 + NB:]

        cid = lax.axis_index("c")
        sid = lax.axis_index("s")
        wid = sid * _NC + cid
        rbase = sid * _RPS
        ebase = wid * _EPW

        _zero_rows(bufs[0], d)
        for r in range(_RPS // _K):
            pltpu.sync_copy(bufs[0], acc.at[pl.ds(rbase + r * _K, _K)])
        pltpu.sync_copy(ei_hbm.at[1, wid], dstv)
        if gather:
            pltpu.sync_copy(ei_hbm.at[0, wid], srcv)
        plsc.subcore_barrier()

        def fetch(j, buf, sem):
            if gather:
                pltpu.async_copy(tab_hbm.at[srcv.at[j]], buf, sem)
            else:
                pltpu.async_copy(
                    tab_hbm.at[pl.ds(ebase + j * _K, _K)], buf, sem)

        def fetch_wait(buf, sem):
            if gather:
                pltpu.make_async_copy(
                    tab_hbm.at[srcv.at[0]], buf, sem).wait()
            else:
                pltpu.make_async_copy(
                    tab_hbm.at[pl.ds(ebase, _K)], buf, sem).wait()

        for p in range(NB):
            fetch(p, bufs[p], gsem[p])

        niter = (_CH - 1) // NB

        @pl.loop(0, niter)
        def _(i):
            j0 = i * NB
            hs = []
            for p in range(NB):
                fetch_wait(bufs[p], gsem[p])
                hs.append(pltpu.async_copy(
                    bufs[p], acc.at[dstv.at[j0 + p]], ssem[p], add=True))
            for p in range(NB):
                hs[p].wait()
                nxt = j0 + p + NB
                @pl.when(nxt < _CH)
                def _():
                    fetch(nxt, bufs[p], gsem[p])

        for p in range(_CH - niter * NB):
            fetch_wait(bufs[p], gsem[p])
            pltpu.async_copy(
                bufs[p], acc.at[dstv.at[niter * NB + p]], ssem[p],
                add=True).wait()

        plsc.subcore_barrier()
        pltpu.sync_copy(acc.at[pl.ds(rbase, _RPS)],
                        out_hbm.at[cid, pl.ds(rbase, _RPS)])

    return body(table, ei4)[0]


def _tc_edge_pad(edge_attr):
    blk = 8000
    def body(ea_ref, o_ref):
        ea = ea_ref[...]
        o_ref[...] = jnp.concatenate(
            [ea,
             jnp.ones((blk, 1), _f32),
             jnp.zeros((blk, _DE - 3), _f32)], axis=1)
    return pl.pallas_call(
        body,
        grid=(_E // blk,),
        in_specs=[pl.BlockSpec((blk, 2), lambda i: (i, 0))],
        out_specs=pl.BlockSpec((blk, _DE), lambda i: (i, 0)),
        out_shape=jax.ShapeDtypeStruct((_E, _DE), _f32))(edge_attr)


def _tc_in_matmul(x, w1):
    def body(x_ref, w_ref, o_ref):
        w = jnp.concatenate(
            [w_ref[...], jnp.zeros((w_ref.shape[0], _D1P - w_ref.shape[1]),
                                   _f32)], axis=1)
        y = jnp.dot(x_ref[...], w, preferred_element_type=_f32)
        o_ref[...] = jnp.concatenate(
            [y, jnp.zeros((_NP - _N, _D1P), _f32)], axis=0)
    return pl.pallas_call(
        body, out_shape=jax.ShapeDtypeStruct((_NP, _D1P), _f32))(x, w1)


def _tc_epilogue1(s1p, ecp, we1, w1, b1, w2):
    def body(s_ref, ec_ref, we_ref, w1_ref, b_ref, w2_ref, o_ref):
        s = s_ref[0] + s_ref[1]
        ec = ec_ref[0] + ec_ref[1]
        w1p = jnp.concatenate(
            [w1_ref[...], jnp.zeros((128, _D1P - 69), _f32)], axis=1)
        we = jnp.concatenate(
            [we_ref[...], jnp.zeros((_DE - 2, 128), _f32)], axis=0)
        t = jnp.dot(we, w1p, preferred_element_type=_f32)
        lane = lax.broadcasted_iota(jnp.int32, (_NP, _DE), 1)
        ecz = jnp.where(lane == 2, 0.0, ec)
        cterm = jnp.dot(ecz, t, preferred_element_type=_f32)
        deg = jnp.maximum(ec[:, 2:3], 1.0)
        bp = jnp.concatenate(
            [b_ref[...], jnp.zeros((1, _D1P - 69), _f32)], axis=1)
        h = jnp.maximum((s + cterm) / deg + bp, 0.0)
        w2p = jnp.concatenate(
            [jnp.concatenate([w2_ref[...], jnp.zeros((69, _D2P - 10), _f32)],
                             axis=1),
             jnp.zeros((_D1P - 69, _D2P), _f32)], axis=0)
        o_ref[...] = jnp.dot(h, w2p, preferred_element_type=_f32)

    return pl.pallas_call(
        body, out_shape=jax.ShapeDtypeStruct((_NP, _D2P), _f32))(
            s1p, ecp, we1, w1, b1.reshape(1, 69), w2)


def _tc_epilogue2(s2p, ecp, we2, w2, b2):
    def body(s_ref, ec_ref, we_ref, w2_ref, b_ref, o_ref):
        s = s_ref[0] + s_ref[1]
        ec = ec_ref[0] + ec_ref[1]
        we = jnp.concatenate(
            [we_ref[...], jnp.zeros((_DE - 2, 69), _f32)], axis=0)
        t = jnp.dot(we, w2_ref[...], preferred_element_type=_f32)
        lane = lax.broadcasted_iota(jnp.int32, (_NP, _DE), 1)
        ecz = jnp.where(lane == 2, 0.0, ec)
        cterm = jnp.dot(ecz, t, preferred_element_type=_f32)
        deg = jnp.maximum(ec[:, 2:3], 1.0)
        h = jnp.maximum((s[:, :10] + cterm) / deg + b_ref[...], 0.0)
        o_ref[...] = h[:_N]

    return pl.pallas_call(
        body, out_shape=jax.ShapeDtypeStruct((_N, 10), _f32))(
            s2p, ecp, we2, w2, b2.reshape(1, 10))


def kernel(x, edge_index, edge_attr, We1, W1, b1, We2, W2, b2):
    ei4 = edge_index.reshape(2, _NW, _CH, _K)
    y1 = _tc_in_matmul(x, W1)
    s1p = _sc_scatter(y1, ei4, _D1P, True)
    eap = _tc_edge_pad(edge_attr)
    ecp = _sc_scatter(eap, ei4, _DE, False)
    y2 = _tc_epilogue1(s1p, ecp, We1, W1, b1, W2)
    s2p = _sc_scatter(y2, ei4, _D2P, True)
    return _tc_epilogue2(s2p, ecp, We2, W2, b2)


# speedup vs baseline: 8.1962x; 1.0796x over previous
import functools

import jax
import jax.numpy as jnp
from jax import lax
from jax.experimental import pallas as pl
from jax.experimental.pallas import tpu as pltpu
from jax.experimental.pallas import tpu_sc as plsc

_N = 10000
_NP = 10240
_E = 320000
_NC = 2
_NS = 16
_NW = _NC * _NS
_K = 80
_EPW = _E // _NW
_CH = _EPW // _K
_RPS = _NP // _NS
_D1P = 80
_DE = 16
_D2P = 16

_f32 = jnp.float32


def _sc_mesh():
    return plsc.VectorSubcoreMesh(core_axis_name="c", subcore_axis_name="s")


def _zero_rows(zb, ncols):
    @pl.loop(0, zb.shape[0])
    def _(r):
        for c in range(ncols // 16):
            zb[r, pl.ds(c * 16, 16)] = jnp.zeros((16,), _f32)


def _sc_scatter(table, src3d, dst3d, d, gather):
    NB = 4
    scratch = [pltpu.VMEM((_CH, _K), jnp.int32)]
    if gather:
        scratch.append(pltpu.VMEM((_CH, _K), jnp.int32))
    scratch += [pltpu.VMEM((_K, d), _f32) for _ in range(NB)]
    scratch += [pltpu.VMEM_SHARED((_NP, d), _f32)]
    scratch += [pltpu.SemaphoreType.DMA] * (2 * NB)

    @functools.partial(
        pl.kernel,
        mesh=_sc_mesh(),
        compiler_params=pltpu.CompilerParams(use_tc_tiling_on_sc=False),
        out_type=[jax.ShapeDtypeStruct((_NC, _NP, d), _f32)],
        scratch_types=scratch,
    )
    def body(tab_hbm, *rest):
        if gather:
            src_hbm, dst_hbm, out_hbm = rest[0], rest[1], rest[2]
            rest = rest[3:]
        else:
            dst_hbm, out_hbm = rest[0], rest[1]
            rest = rest[2:]
        dstv = rest[0]
        rest = rest[1:]
        if gather:
            srcv = rest[0]
            rest = rest[1:]
        bufs = rest[:NB]
        acc = rest[NB]
        gsem = rest[NB + 1:NB + 1 + NB]
        ssem = rest[NB + 1 + NB:]

        cid = lax.axis_index("c")
        sid = lax.axis_index("s")
        wid = sid * _NC + cid
        rbase = sid * _RPS
        ebase = wid * _EPW

        _zero_rows(bufs[0], d)
        for r in range(_RPS // _K):
            pltpu.sync_copy(bufs[0], acc.at[pl.ds(rbase + r * _K, _K)])
        pltpu.sync_copy(dst_hbm.at[wid], dstv)
        if gather:
            pltpu.sync_copy(src_hbm.at[wid], srcv)
        plsc.subcore_barrier()

        def fetch(j, buf, sem):
            if gather:
                pltpu.async_copy(tab_hbm.at[srcv.at[j]], buf, sem)
            else:
                pltpu.async_copy(
                    tab_hbm.at[pl.ds(ebase + j * _K, _K)], buf, sem)

        def fetch_wait(buf, sem):
            if gather:
                pltpu.make_async_copy(
                    tab_hbm.at[srcv.at[0]], buf, sem).wait()
            else:
                pltpu.make_async_copy(
                    tab_hbm.at[pl.ds(ebase, _K)], buf, sem).wait()

        for p in range(NB):
            fetch(p, bufs[p], gsem[p])

        niter = (_CH - 1) // NB

        @pl.loop(0, niter)
        def _(i):
            j0 = i * NB
            hs = []
            for p in range(NB):
                fetch_wait(bufs[p], gsem[p])
                hs.append(pltpu.async_copy(
                    bufs[p], acc.at[dstv.at[j0 + p]], ssem[p], add=True))
            for p in range(NB):
                hs[p].wait()
                nxt = j0 + p + NB
                @pl.when(nxt < _CH)
                def _():
                    fetch(nxt, bufs[p], gsem[p])

        for p in range(_CH - niter * NB):
            fetch_wait(bufs[p], gsem[p])
            pltpu.async_copy(
                bufs[p], acc.at[dstv.at[niter * NB + p]], ssem[p],
                add=True).wait()

        plsc.subcore_barrier()
        pltpu.sync_copy(acc.at[pl.ds(rbase, _RPS)],
                        out_hbm.at[cid, pl.ds(rbase, _RPS)])

    if gather:
        return body(table, src3d, dst3d)[0]
    return body(table, dst3d)[0]


def _tc_ei_repack(edge_index):
    def body(ei_ref, s_ref, d_ref):
        ei = ei_ref[...]
        s_ref[...] = ei[0]
        d_ref[...] = ei[1]
    return pl.pallas_call(
        body,
        out_shape=[jax.ShapeDtypeStruct((_E,), jnp.int32),
                   jax.ShapeDtypeStruct((_E,), jnp.int32)],
    )(edge_index)


def _tc_edge_pad(edge_attr, y1):
    blk = 8000
    def body(ea_ref, y_ref, o_ref):
        ea = ea_ref[...]
        o_ref[...] = jnp.concatenate(
            [ea,
             jnp.ones((blk, 1), _f32),
             jnp.zeros((blk, _DE - 3), _f32)], axis=1)
    return pl.pallas_call(
        body,
        grid=(_E // blk,),
        in_specs=[pl.BlockSpec((blk, 2), lambda i: (i, 0)),
                  pl.BlockSpec((8, _D1P), lambda i: (0, 0))],
        out_specs=pl.BlockSpec((blk, _DE), lambda i: (i, 0)),
        out_shape=jax.ShapeDtypeStruct((_E, _DE), _f32))(edge_attr, y1)


def _tc_in_matmul(x, w1):
    def body(x_ref, w_ref, o_ref):
        w = jnp.concatenate(
            [w_ref[...], jnp.zeros((w_ref.shape[0], _D1P - w_ref.shape[1]),
                                   _f32)], axis=1)
        y = jnp.dot(x_ref[...], w, preferred_element_type=_f32)
        o_ref[...] = jnp.concatenate(
            [y, jnp.zeros((_NP - _N, _D1P), _f32)], axis=0)
    return pl.pallas_call(
        body, out_shape=jax.ShapeDtypeStruct((_NP, _D1P), _f32))(x, w1)


def _tc_epilogue1(s1p, ecp, we1, w1, b1, w2):
    def body(s_ref, ec_ref, we_ref, w1_ref, b_ref, w2_ref, o_ref):
        s = s_ref[0] + s_ref[1]
        ec = ec_ref[0] + ec_ref[1]
        w1p = jnp.concatenate(
            [w1_ref[...], jnp.zeros((128, _D1P - 69), _f32)], axis=1)
        we = jnp.concatenate(
            [we_ref[...], jnp.zeros((_DE - 2, 128), _f32)], axis=0)
        t = jnp.dot(we, w1p, preferred_element_type=_f32)
        lane = lax.broadcasted_iota(jnp.int32, (_NP, _DE), 1)
        ecz = jnp.where(lane == 2, 0.0, ec)
        cterm = jnp.dot(ecz, t, preferred_element_type=_f32)
        deg = jnp.maximum(ec[:, 2:3], 1.0)
        bp = jnp.concatenate(
            [b_ref[...], jnp.zeros((1, _D1P - 69), _f32)], axis=1)
        h = jnp.maximum((s + cterm) / deg + bp, 0.0)
        w2p = jnp.concatenate(
            [jnp.concatenate([w2_ref[...], jnp.zeros((69, _D2P - 10), _f32)],
                             axis=1),
             jnp.zeros((_D1P - 69, _D2P), _f32)], axis=0)
        o_ref[...] = jnp.dot(h, w2p, preferred_element_type=_f32)

    return pl.pallas_call(
        body, out_shape=jax.ShapeDtypeStruct((_NP, _D2P), _f32))(
            s1p, ecp, we1, w1, b1.reshape(1, 69), w2)


def _tc_epilogue2(s2p, ecp, we2, w2, b2):
    def body(s_ref, ec_ref, we_ref, w2_ref, b_ref, o_ref):
        s = s_ref[0] + s_ref[1]
        ec = ec_ref[0] + ec_ref[1]
        we = jnp.concatenate(
            [we_ref[...], jnp.zeros((_DE - 2, 69), _f32)], axis=0)
        t = jnp.dot(we, w2_ref[...], preferred_element_type=_f32)
        lane = lax.broadcasted_iota(jnp.int32, (_NP, _DE), 1)
        ecz = jnp.where(lane == 2, 0.0, ec)
        cterm = jnp.dot(ecz, t, preferred_element_type=_f32)
        deg = jnp.maximum(ec[:, 2:3], 1.0)
        h = jnp.maximum((s[:, :10] + cterm) / deg + b_ref[...], 0.0)
        o_ref[...] = h[:_N]

    return pl.pallas_call(
        body, out_shape=jax.ShapeDtypeStruct((_N, 10), _f32))(
            s2p, ecp, we2, w2, b2.reshape(1, 10))


def kernel(x, edge_index, edge_attr, We1, W1, b1, We2, W2, b2):
    srcl, dstl = _tc_ei_repack(edge_index)
    src3d = srcl.reshape(_NW, _CH, _K)
    dst3d = dstl.reshape(_NW, _CH, _K)
    y1 = _tc_in_matmul(x, W1)
    s1p = _sc_scatter(y1, src3d, dst3d, _D1P, True)
    eap = _tc_edge_pad(edge_attr, y1)
    ecp = _sc_scatter(eap, src3d, dst3d, _DE, False)
    y2 = _tc_epilogue1(s1p, ecp, We1, W1, b1, W2)
    s2p = _sc_scatter(y2, src3d, dst3d, _D2P, True)
    return _tc_epilogue2(s2p, ecp, We2, W2, b2)


# speedup vs baseline: 17.8226x; 2.1745x over previous
import dataclasses
import functools

import jax
import jax.numpy as jnp
from jax import lax
from jax.experimental import pallas as pl
from jax.experimental.pallas import tpu as pltpu
from jax.experimental.pallas import tpu_sc as plsc

_N = 10000
_NP = 10240
_E = 320000
_NC = 2
_NS = 16
_NW = _NC * _NS
_K = 80
_EPW = _E // _NW
_CH = _EPW // _K
_RPS = _NP // _NS
_D1P = 80
_DE = 16
_D2P = 16

_f32 = jnp.float32


def _sc_mesh():
    return plsc.VectorSubcoreMesh(core_axis_name="c", subcore_axis_name="s")


def _sc_params(layout_passes=True):
    cp = pltpu.CompilerParams(use_tc_tiling_on_sc=False)
    if not layout_passes and (
            "needs_layout_passes" in pltpu.CompilerParams.__dataclass_fields__):
        cp = dataclasses.replace(cp, needs_layout_passes=False)
    return cp


def _zero_rows(zb, ncols):
    @pl.loop(0, zb.shape[0])
    def _(r):
        for c in range(ncols // 16):
            zb[r, pl.ds(c * 16, 16)] = jnp.zeros((16,), _f32)


def _sc_scatter(table, src3d, dst3d, d, gather):
    NB = 4
    scratch = [pltpu.VMEM((_CH, _K), jnp.int32)]
    if gather:
        scratch.append(pltpu.VMEM((_CH, _K), jnp.int32))
    scratch += [pltpu.VMEM((_K, d), _f32) for _ in range(NB)]
    scratch += [pltpu.VMEM_SHARED((_NP, d), _f32)]
    scratch += [pltpu.SemaphoreType.DMA] * (2 * NB)

    @functools.partial(
        pl.kernel,
        mesh=_sc_mesh(),
        compiler_params=_sc_params(),
        out_type=[jax.ShapeDtypeStruct((_NC, _NP, d), _f32)],
        scratch_types=scratch,
    )
    def body(tab_hbm, *rest):
        if gather:
            src_hbm, dst_hbm, out_hbm = rest[0], rest[1], rest[2]
            rest = rest[3:]
        else:
            dst_hbm, out_hbm = rest[0], rest[1]
            rest = rest[2:]
        dstv = rest[0]
        rest = rest[1:]
        if gather:
            srcv = rest[0]
            rest = rest[1:]
        bufs = rest[:NB]
        acc = rest[NB]
        gsem = rest[NB + 1:NB + 1 + NB]
        ssem = rest[NB + 1 + NB:]

        cid = lax.axis_index("c")
        sid = lax.axis_index("s")
        wid = sid * _NC + cid
        rbase = sid * _RPS
        ebase = wid * _EPW

        _zero_rows(bufs[0], d)
        for r in range(_RPS // _K):
            pltpu.sync_copy(bufs[0], acc.at[pl.ds(rbase + r * _K, _K)])
        pltpu.sync_copy(dst_hbm.at[wid], dstv)
        if gather:
            pltpu.sync_copy(src_hbm.at[wid], srcv)
        plsc.subcore_barrier()

        def fetch(j, buf, sem):
            if gather:
                pltpu.async_copy(tab_hbm.at[srcv.at[j]], buf, sem)
            else:
                pltpu.async_copy(
                    tab_hbm.at[pl.ds(ebase + j * _K, _K)], buf, sem)

        def fetch_wait(buf, sem):
            if gather:
                pltpu.make_async_copy(
                    tab_hbm.at[srcv.at[0]], buf, sem).wait()
            else:
                pltpu.make_async_copy(
                    tab_hbm.at[pl.ds(ebase, _K)], buf, sem).wait()

        for p in range(NB):
            fetch(p, bufs[p], gsem[p])

        niter = (_CH - 1) // NB

        @pl.loop(0, niter)
        def _(i):
            j0 = i * NB
            hs = []
            for p in range(NB):
                fetch_wait(bufs[p], gsem[p])
                hs.append(pltpu.async_copy(
                    bufs[p], acc.at[dstv.at[j0 + p]], ssem[p], add=True))
            for p in range(NB):
                hs[p].wait()
                nxt = j0 + p + NB
                @pl.when(nxt < _CH)
                def _():
                    fetch(nxt, bufs[p], gsem[p])

        for p in range(_CH - niter * NB):
            fetch_wait(bufs[p], gsem[p])
            pltpu.async_copy(
                bufs[p], acc.at[dstv.at[niter * NB + p]], ssem[p],
                add=True).wait()

        plsc.subcore_barrier()
        pltpu.sync_copy(acc.at[pl.ds(rbase, _RPS)],
                        out_hbm.at[cid, pl.ds(rbase, _RPS)])

    if gather:
        return body(table, src3d, dst3d)[0]
    return body(table, dst3d)[0]


def _sc_scatter_edges(ea0, ea1, dst3d):
    NB = 2
    scratch = [
        pltpu.VMEM((_CH, _K), jnp.int32),
        pltpu.VMEM((_EPW,), _f32),
        pltpu.VMEM((_EPW,), _f32),
        pltpu.VMEM((_K, _DE), _f32),
        pltpu.VMEM((_K, _DE), _f32),
        pltpu.VMEM_SHARED((_NP, _DE), _f32),
        pltpu.SemaphoreType.DMA,
        pltpu.SemaphoreType.DMA,
    ]

    @functools.partial(
        pl.kernel,
        mesh=_sc_mesh(),
        compiler_params=_sc_params(layout_passes=False),
        out_type=[jax.ShapeDtypeStruct((_NC, _NP, _DE), _f32)],
        scratch_types=scratch,
    )
    def body(ea0_hbm, ea1_hbm, dst_hbm, out_hbm,
             dstv, a0, a1, b0, b1, acc, s0, s1):
        bufs = (b0, b1)
        ssem = (s0, s1)
        cid = lax.axis_index("c")
        sid = lax.axis_index("s")
        wid = sid * _NC + cid
        rbase = sid * _RPS
        ebase = wid * _EPW

        pltpu.sync_copy(dst_hbm.at[wid], dstv)
        pltpu.sync_copy(ea0_hbm.at[pl.ds(ebase, _EPW)], a0)
        pltpu.sync_copy(ea1_hbm.at[pl.ds(ebase, _EPW)], a1)

        lane = lax.iota(jnp.int32, 16)
        zero16 = jnp.zeros((16,), _f32)
        ones16 = zero16 + 1.0

        @pl.loop(0, _K)
        def _(r):
            rv = jnp.zeros((16,), jnp.int32) + r
            plsc.store_scatter(b0, [rv, lane], zero16)
            plsc.store_scatter(b1, [rv, lane], zero16)
        for r in range(_RPS // _K):
            pltpu.sync_copy(b0, acc.at[pl.ds(rbase + r * _K, _K)])
        col2 = jnp.zeros((16,), jnp.int32) + 2
        for r in range(_K // 16):
            ridx = lane + 16 * r
            plsc.store_scatter(b0, [ridx, col2], ones16)
            plsc.store_scatter(b1, [ridx, col2], ones16)
        plsc.subcore_barrier()

        col0 = jnp.zeros((16,), jnp.int32)
        col1 = col0 + 1

        def build(j, buf):
            for r in range(_K // 16):
                ridx = lane + 16 * r
                off = j * _K + 16 * r
                plsc.store_scatter(buf, [ridx, col0], a0[pl.ds(off, 16)])
                plsc.store_scatter(buf, [ridx, col1], a1[pl.ds(off, 16)])

        def swait(p):
            pltpu.make_async_copy(
                bufs[p], acc.at[dstv.at[0]], ssem[p]).wait()

        @pl.loop(0, _CH // NB)
        def _(i):
            for p in range(NB):
                j = i * NB + p
                @pl.when(i > 0)
                def _():
                    swait(p)
                build(j, bufs[p])
                pltpu.async_copy(bufs[p], acc.at[dstv.at[j]], ssem[p],
                                 add=True)

        for p in range(_CH - (_CH // NB) * NB):
            swait(p)
            build((_CH // NB) * NB + p, bufs[p])
            pltpu.async_copy(bufs[p], acc.at[dstv.at[(_CH // NB) * NB + p]],
                             ssem[p], add=True)
        for p in range(NB):
            swait(p)

        plsc.subcore_barrier()
        pltpu.sync_copy(acc.at[pl.ds(rbase, _RPS)],
                        out_hbm.at[cid, pl.ds(rbase, _RPS)])

    return body(ea0, ea1, dst3d)[0]


def _tc_split2(arr2e):
    def body(a_ref, r0_ref, r1_ref):
        a = a_ref[...]
        r0_ref[...] = a[0]
        r1_ref[...] = a[1]
    dt = arr2e.dtype
    return pl.pallas_call(
        body,
        out_shape=[jax.ShapeDtypeStruct((_E,), dt),
                   jax.ShapeDtypeStruct((_E,), dt)],
    )(arr2e)


def _tc_in_matmul(x, w1):
    def body(x_ref, w_ref, o_ref):
        w = jnp.concatenate(
            [w_ref[...], jnp.zeros((w_ref.shape[0], _D1P - w_ref.shape[1]),
                                   _f32)], axis=1)
        y = jnp.dot(x_ref[...], w, preferred_element_type=_f32)
        o_ref[...] = jnp.concatenate(
            [y, jnp.zeros((_NP - _N, _D1P), _f32)], axis=0)
    return pl.pallas_call(
        body, out_shape=jax.ShapeDtypeStruct((_NP, _D1P), _f32))(x, w1)


def _tc_epilogue1(s1p, ecp, we1, w1, b1, w2):
    def body(s_ref, ec_ref, we_ref, w1_ref, b_ref, w2_ref, o_ref):
        s = s_ref[0] + s_ref[1]
        ec = ec_ref[0] + ec_ref[1]
        w1p = jnp.concatenate(
            [w1_ref[...], jnp.zeros((128, _D1P - 69), _f32)], axis=1)
        we = jnp.concatenate(
            [we_ref[...], jnp.zeros((_DE - 2, 128), _f32)], axis=0)
        t = jnp.dot(we, w1p, preferred_element_type=_f32)
        lane = lax.broadcasted_iota(jnp.int32, (_NP, _DE), 1)
        ecz = jnp.where(lane == 2, 0.0, ec)
        cterm = jnp.dot(ecz, t, preferred_element_type=_f32)
        deg = jnp.maximum(ec[:, 2:3], 1.0)
        bp = jnp.concatenate(
            [b_ref[...], jnp.zeros((1, _D1P - 69), _f32)], axis=1)
        h = jnp.maximum((s + cterm) / deg + bp, 0.0)
        w2p = jnp.concatenate(
            [jnp.concatenate([w2_ref[...], jnp.zeros((69, _D2P - 10), _f32)],
                             axis=1),
             jnp.zeros((_D1P - 69, _D2P), _f32)], axis=0)
        o_ref[...] = jnp.dot(h, w2p, preferred_element_type=_f32)

    return pl.pallas_call(
        body, out_shape=jax.ShapeDtypeStruct((_NP, _D2P), _f32))(
            s1p, ecp, we1, w1, b1.reshape(1, 69), w2)


def _tc_epilogue2(s2p, ecp, we2, w2, b2):
    def body(s_ref, ec_ref, we_ref, w2_ref, b_ref, o_ref):
        s = s_ref[0] + s_ref[1]
        ec = ec_ref[0] + ec_ref[1]
        we = jnp.concatenate(
            [we_ref[...], jnp.zeros((_DE - 2, 69), _f32)], axis=0)
        t = jnp.dot(we, w2_ref[...], preferred_element_type=_f32)
        lane = lax.broadcasted_iota(jnp.int32, (_NP, _DE), 1)
        ecz = jnp.where(lane == 2, 0.0, ec)
        cterm = jnp.dot(ecz, t, preferred_element_type=_f32)
        deg = jnp.maximum(ec[:, 2:3], 1.0)
        h = jnp.maximum((s[:, :10] + cterm) / deg + b_ref[...], 0.0)
        o_ref[...] = h[:_N]

    return pl.pallas_call(
        body, out_shape=jax.ShapeDtypeStruct((_N, 10), _f32))(
            s2p, ecp, we2, w2, b2.reshape(1, 10))


def kernel(x, edge_index, edge_attr, We1, W1, b1, We2, W2, b2):
    srcl, dstl = _tc_split2(edge_index)
    src3d = srcl.reshape(_NW, _CH, _K)
    dst3d = dstl.reshape(_NW, _CH, _K)
    ea0, ea1 = _tc_split2(edge_attr.T)
    y1 = _tc_in_matmul(x, W1)
    s1p = _sc_scatter(y1, src3d, dst3d, _D1P, True)
    ecp = _sc_scatter_edges(ea0, ea1, dst3d)
    y2 = _tc_epilogue1(s1p, ecp, We1, W1, b1, W2)
    s2p = _sc_scatter(y2, src3d, dst3d, _D2P, True)
    return _tc_epilogue2(s2p, ecp, We2, W2, b2)


# speedup vs baseline: 19.0904x; 1.0711x over previous
import dataclasses
import functools

import jax
import jax.numpy as jnp
from jax import lax
from jax.experimental import pallas as pl
from jax.experimental.pallas import tpu as pltpu
from jax.experimental.pallas import tpu_sc as plsc

_N = 10000
_NP = 10240
_E = 320000
_NC = 2
_NS = 16
_NW = _NC * _NS
_K = 80
_EPW = _E // _NW
_CH = _EPW // _K
_RPS = _NP // _NS
_D1P = 80
_DE = 16
_D2P = 16

_f32 = jnp.float32


def _sc_mesh():
    return plsc.VectorSubcoreMesh(core_axis_name="c", subcore_axis_name="s")


def _sc_params(layout_passes=True):
    cp = pltpu.CompilerParams(use_tc_tiling_on_sc=False)
    if not layout_passes and (
            "needs_layout_passes" in pltpu.CompilerParams.__dataclass_fields__):
        cp = dataclasses.replace(cp, needs_layout_passes=False)
    return cp


def _zero_rows(zb, ncols):
    @pl.loop(0, zb.shape[0])
    def _(r):
        for c in range(ncols // 16):
            zb[r, pl.ds(c * 16, 16)] = jnp.zeros((16,), _f32)


def _sc_scatter(table, src3d, dst3d, d, gather):
    NB = 6 if d > 16 else 8
    scratch = [pltpu.VMEM((_CH, _K), jnp.int32)]
    if gather:
        scratch.append(pltpu.VMEM((_CH, _K), jnp.int32))
    scratch += [pltpu.VMEM((_K, d), _f32) for _ in range(NB)]
    scratch += [pltpu.VMEM_SHARED((_NP, d), _f32)]
    scratch += [pltpu.SemaphoreType.DMA] * (2 * NB)

    @functools.partial(
        pl.kernel,
        mesh=_sc_mesh(),
        compiler_params=_sc_params(),
        out_type=[jax.ShapeDtypeStruct((_NC, _NP, d), _f32)],
        scratch_types=scratch,
    )
    def body(tab_hbm, *rest):
        if gather:
            src_hbm, dst_hbm, out_hbm = rest[0], rest[1], rest[2]
            rest = rest[3:]
        else:
            dst_hbm, out_hbm = rest[0], rest[1]
            rest = rest[2:]
        dstv = rest[0]
        rest = rest[1:]
        if gather:
            srcv = rest[0]
            rest = rest[1:]
        bufs = rest[:NB]
        acc = rest[NB]
        gsem = rest[NB + 1:NB + 1 + NB]
        ssem = rest[NB + 1 + NB:]

        cid = lax.axis_index("c")
        sid = lax.axis_index("s")
        wid = sid * _NC + cid
        rbase = sid * _RPS
        ebase = wid * _EPW

        _zero_rows(bufs[0], d)
        for r in range(_RPS // _K):
            pltpu.sync_copy(bufs[0], acc.at[pl.ds(rbase + r * _K, _K)])
        pltpu.sync_copy(dst_hbm.at[wid], dstv)
        if gather:
            pltpu.sync_copy(src_hbm.at[wid], srcv)
        plsc.subcore_barrier()

        def fetch(j, buf, sem):
            if gather:
                pltpu.async_copy(tab_hbm.at[srcv.at[j]], buf, sem)
            else:
                pltpu.async_copy(
                    tab_hbm.at[pl.ds(ebase + j * _K, _K)], buf, sem)

        def fetch_wait(buf, sem):
            if gather:
                pltpu.make_async_copy(
                    tab_hbm.at[srcv.at[0]], buf, sem).wait()
            else:
                pltpu.make_async_copy(
                    tab_hbm.at[pl.ds(ebase, _K)], buf, sem).wait()

        for p in range(NB):
            fetch(p, bufs[p], gsem[p])

        niter = (_CH - 1) // NB

        @pl.loop(0, niter)
        def _(i):
            j0 = i * NB
            hs = []
            for p in range(NB):
                fetch_wait(bufs[p], gsem[p])
                hs.append(pltpu.async_copy(
                    bufs[p], acc.at[dstv.at[j0 + p]], ssem[p], add=True))
            for p in range(NB):
                hs[p].wait()
                nxt = j0 + p + NB
                @pl.when(nxt < _CH)
                def _():
                    fetch(nxt, bufs[p], gsem[p])

        for p in range(_CH - niter * NB):
            fetch_wait(bufs[p], gsem[p])
            pltpu.async_copy(
                bufs[p], acc.at[dstv.at[niter * NB + p]], ssem[p],
                add=True).wait()

        plsc.subcore_barrier()
        pltpu.sync_copy(acc.at[pl.ds(rbase, _RPS)],
                        out_hbm.at[cid, pl.ds(rbase, _RPS)])

    if gather:
        return body(table, src3d, dst3d)[0]
    return body(table, dst3d)[0]


def _sc_scatter_edges(ea0, ea1, dst3d):
    NB = 2
    scratch = [
        pltpu.VMEM((_CH, _K), jnp.int32),
        pltpu.VMEM((_EPW,), _f32),
        pltpu.VMEM((_EPW,), _f32),
        pltpu.VMEM((_K, _DE), _f32),
        pltpu.VMEM((_K, _DE), _f32),
        pltpu.VMEM_SHARED((_NP, _DE), _f32),
        pltpu.SemaphoreType.DMA,
        pltpu.SemaphoreType.DMA,
    ]

    @functools.partial(
        pl.kernel,
        mesh=_sc_mesh(),
        compiler_params=_sc_params(layout_passes=False),
        out_type=[jax.ShapeDtypeStruct((_NC, _NP, _DE), _f32)],
        scratch_types=scratch,
    )
    def body(ea0_hbm, ea1_hbm, dst_hbm, out_hbm,
             dstv, a0, a1, b0, b1, acc, s0, s1):
        bufs = (b0, b1)
        ssem = (s0, s1)
        cid = lax.axis_index("c")
        sid = lax.axis_index("s")
        wid = sid * _NC + cid
        rbase = sid * _RPS
        ebase = wid * _EPW

        pltpu.sync_copy(dst_hbm.at[wid], dstv)
        pltpu.sync_copy(ea0_hbm.at[pl.ds(ebase, _EPW)], a0)
        pltpu.sync_copy(ea1_hbm.at[pl.ds(ebase, _EPW)], a1)

        lane = lax.iota(jnp.int32, 16)
        zero16 = jnp.zeros((16,), _f32)
        ones16 = zero16 + 1.0

        @pl.loop(0, _K)
        def _(r):
            rv = jnp.zeros((16,), jnp.int32) + r
            plsc.store_scatter(b0, [rv, lane], zero16)
            plsc.store_scatter(b1, [rv, lane], zero16)
        for r in range(_RPS // _K):
            pltpu.sync_copy(b0, acc.at[pl.ds(rbase + r * _K, _K)])
        col2 = jnp.zeros((16,), jnp.int32) + 2
        for r in range(_K // 16):
            ridx = lane + 16 * r
            plsc.store_scatter(b0, [ridx, col2], ones16)
            plsc.store_scatter(b1, [ridx, col2], ones16)
        plsc.subcore_barrier()

        col0 = jnp.zeros((16,), jnp.int32)
        col1 = col0 + 1

        def build(j, buf):
            for r in range(_K // 16):
                ridx = lane + 16 * r
                off = j * _K + 16 * r
                plsc.store_scatter(buf, [ridx, col0], a0[pl.ds(off, 16)])
                plsc.store_scatter(buf, [ridx, col1], a1[pl.ds(off, 16)])

        def swait(p):
            pltpu.make_async_copy(
                bufs[p], acc.at[dstv.at[0]], ssem[p]).wait()

        @pl.loop(0, _CH // NB)
        def _(i):
            for p in range(NB):
                j = i * NB + p
                @pl.when(i > 0)
                def _():
                    swait(p)
                build(j, bufs[p])
                pltpu.async_copy(bufs[p], acc.at[dstv.at[j]], ssem[p],
                                 add=True)

        for p in range(_CH - (_CH // NB) * NB):
            swait(p)
            build((_CH // NB) * NB + p, bufs[p])
            pltpu.async_copy(bufs[p], acc.at[dstv.at[(_CH // NB) * NB + p]],
                             ssem[p], add=True)
        for p in range(NB):
            swait(p)

        plsc.subcore_barrier()
        pltpu.sync_copy(acc.at[pl.ds(rbase, _RPS)],
                        out_hbm.at[cid, pl.ds(rbase, _RPS)])

    return body(ea0, ea1, dst3d)[0]


def _tc_split2(arr2e):
    def body(a_ref, r0_ref, r1_ref):
        a = a_ref[...]
        r0_ref[...] = a[0]
        r1_ref[...] = a[1]
    dt = arr2e.dtype
    return pl.pallas_call(
        body,
        out_shape=[jax.ShapeDtypeStruct((_E,), dt),
                   jax.ShapeDtypeStruct((_E,), dt)],
    )(arr2e)


def _tc_in_matmul(x, w1):
    def body(x_ref, w_ref, o_ref):
        w = jnp.concatenate(
            [w_ref[...], jnp.zeros((w_ref.shape[0], _D1P - w_ref.shape[1]),
                                   _f32)], axis=1)
        y = jnp.dot(x_ref[...], w, preferred_element_type=_f32)
        o_ref[...] = jnp.concatenate(
            [y, jnp.zeros((_NP - _N, _D1P), _f32)], axis=0)
    return pl.pallas_call(
        body, out_shape=jax.ShapeDtypeStruct((_NP, _D1P), _f32))(x, w1)


def _tc_epilogue1(s1p, ecp, we1, w1, b1, w2):
    def body(s_ref, ec_ref, we_ref, w1_ref, b_ref, w2_ref, o_ref):
        s = s_ref[0] + s_ref[1]
        ec = ec_ref[0] + ec_ref[1]
        w1p = jnp.concatenate(
            [w1_ref[...], jnp.zeros((128, _D1P - 69), _f32)], axis=1)
        we = jnp.concatenate(
            [we_ref[...], jnp.zeros((_DE - 2, 128), _f32)], axis=0)
        t = jnp.dot(we, w1p, preferred_element_type=_f32)
        lane = lax.broadcasted_iota(jnp.int32, (_NP, _DE), 1)
        ecz = jnp.where(lane == 2, 0.0, ec)
        cterm = jnp.dot(ecz, t, preferred_element_type=_f32)
        deg = jnp.maximum(ec[:, 2:3], 1.0)
        bp = jnp.concatenate(
            [b_ref[...], jnp.zeros((1, _D1P - 69), _f32)], axis=1)
        h = jnp.maximum((s + cterm) / deg + bp, 0.0)
        w2p = jnp.concatenate(
            [jnp.concatenate([w2_ref[...], jnp.zeros((69, _D2P - 10), _f32)],
                             axis=1),
             jnp.zeros((_D1P - 69, _D2P), _f32)], axis=0)
        o_ref[...] = jnp.dot(h, w2p, preferred_element_type=_f32)

    return pl.pallas_call(
        body, out_shape=jax.ShapeDtypeStruct((_NP, _D2P), _f32))(
            s1p, ecp, we1, w1, b1.reshape(1, 69), w2)


def _tc_epilogue2(s2p, ecp, we2, w2, b2):
    def body(s_ref, ec_ref, we_ref, w2_ref, b_ref, o_ref):
        s = s_ref[0] + s_ref[1]
        ec = ec_ref[0] + ec_ref[1]
        we = jnp.concatenate(
            [we_ref[...], jnp.zeros((_DE - 2, 69), _f32)], axis=0)
        t = jnp.dot(we, w2_ref[...], preferred_element_type=_f32)
        lane = lax.broadcasted_iota(jnp.int32, (_NP, _DE), 1)
        ecz = jnp.where(lane == 2, 0.0, ec)
        cterm = jnp.dot(ecz, t, preferred_element_type=_f32)
        deg = jnp.maximum(ec[:, 2:3], 1.0)
        h = jnp.maximum((s[:, :10] + cterm) / deg + b_ref[...], 0.0)
        o_ref[...] = h[:_N]

    return pl.pallas_call(
        body, out_shape=jax.ShapeDtypeStruct((_N, 10), _f32))(
            s2p, ecp, we2, w2, b2.reshape(1, 10))


def kernel(x, edge_index, edge_attr, We1, W1, b1, We2, W2, b2):
    srcl, dstl = _tc_split2(edge_index)
    src3d = srcl.reshape(_NW, _CH, _K)
    dst3d = dstl.reshape(_NW, _CH, _K)
    ea0, ea1 = _tc_split2(edge_attr.T)
    y1 = _tc_in_matmul(x, W1)
    ecp = _sc_scatter_edges(ea0, ea1, dst3d)
    s1p = _sc_scatter(y1, src3d, dst3d, _D1P, True)
    y2 = _tc_epilogue1(s1p, ecp, We1, W1, b1, W2)
    s2p = _sc_scatter(y2, src3d, dst3d, _D2P, True)
    return _tc_epilogue2(s2p, ecp, We2, W2, b2)
